# baseline (speedup 1.0000x reference)
import functools
import jax
import jax.numpy as jnp
from jax import lax
from jax.experimental import pallas as pl
from jax.experimental.pallas import tpu as pltpu
from jax.experimental.pallas import tpu_sc as plsc

P = 127
MAX_DEGREE = 8
M = 16
D_MODEL = 128
DEPTH = 8
TOK = DEPTH * M * M
NC, NS, L = 2, 16, 16
NW = NC * NS
CH = 256
NV = D_MODEL // L


def _sc_body(x_hbm, vt_hbm, row_hbm, col_hbm, deg_hbm, out_hbm,
             idx_v, rows_v, row_v, col_v, deg_v, pos_v, gsem):
    wid = lax.axis_index("s") * NC + lax.axis_index("c")
    nrows = out_hbm.shape[0]
    per_w = nrows // NW
    nch = per_w // CH
    wbase = wid * per_w

    pltpu.sync_copy(row_hbm, row_v)
    pltpu.sync_copy(col_hbm, col_v)
    pltpu.sync_copy(deg_hbm, deg_v)
    pltpu.sync_copy(x_hbm.at[pl.ds(wbase, per_w)], idx_v)

    def build_rc(r, _):
        rr = r // M
        cc = lax.rem(r, M)
        for j in range(NV):
            s = pl.ds(j * L, L)
            pos_v[r, s] = row_v[rr, s] + col_v[cc, s]
        return 0
    lax.fori_loop(0, CH, build_rc, 0)

    def chunk_body(k, _):
        base = wbase + k * CH
        d = lax.rem(k, DEPTH)
        pltpu.async_copy(vt_hbm.at[idx_v.at[pl.ds(k * CH, CH)]],
                         rows_v, gsem).wait()

        def add_pos(r, _):
            for j in range(NV):
                s = pl.ds(j * L, L)
                rows_v[r, s] = rows_v[r, s] + (pos_v[r, s] + deg_v[d, s])
            return 0
        lax.fori_loop(0, CH, add_pos, 0)

        pltpu.sync_copy(rows_v, out_hbm.at[pl.ds(base, CH)])
        return 0
    lax.fori_loop(0, nch, chunk_body, 0)


def kernel(x, value_emb, row_emb, col_emb, degree_emb):
    batch = x.shape[0]
    nrows = batch * TOK
    xf = x.reshape(nrows)
    vt = jnp.pad(value_emb, ((0, 1), (0, 0)))
    mesh = plsc.VectorSubcoreMesh(core_axis_name="c", subcore_axis_name="s")
    f = functools.partial(
        pl.kernel, mesh=mesh,
        out_type=jax.ShapeDtypeStruct((nrows, D_MODEL), jnp.float32),
        scratch_types=[
            pltpu.VMEM((nrows // NW,), jnp.int32),
            pltpu.VMEM((CH, D_MODEL), jnp.float32),
            pltpu.VMEM((M, D_MODEL), jnp.float32),
            pltpu.VMEM((M, D_MODEL), jnp.float32),
            pltpu.VMEM((MAX_DEGREE, D_MODEL), jnp.float32),
            pltpu.VMEM((CH, D_MODEL), jnp.float32),
            pltpu.SemaphoreType.DMA,
        ],
    )(_sc_body)
    out = f(xf, vt, row_emb, col_emb, degree_emb)
    return out.reshape(batch, DEPTH, M * M, D_MODEL)


# speedup vs baseline: 1.2787x; 1.2787x over previous
import functools
import jax
import jax.numpy as jnp
from jax import lax
from jax.experimental import pallas as pl
from jax.experimental.pallas import tpu as pltpu
from jax.experimental.pallas import tpu_sc as plsc

P = 127
MAX_DEGREE = 8
M = 16
D_MODEL = 128
DEPTH = 8
TOK = DEPTH * M * M
NC, NS, L = 2, 16, 16
NW = NC * NS
CH = 128
NV = D_MODEL // L
NBUF = 4
BPW = 8


def _sc_body(x_hbm, vt_hbm, row_hbm, col_hbm, deg_hbm, out_hbm,
             idx_v, b0, b1, b2, b3, row_v, col_v, deg_v, pos_v,
             g0, g1, g2, g3, s0, s1, s2, s3):
    wid = lax.axis_index("s") * NC + lax.axis_index("c")
    per_w = BPW * TOK
    nch = per_w // CH
    wbase = wid * per_w
    bufs = [b0, b1, b2, b3]
    gsems = [g0, g1, g2, g3]
    ssems = [s0, s1, s2, s3]

    pltpu.sync_copy(row_hbm, row_v)
    pltpu.sync_copy(col_hbm, col_v)
    pltpu.sync_copy(deg_hbm, deg_v)
    pltpu.sync_copy(x_hbm.at[pl.ds(wbase, per_w)], idx_v)

    def chunk_base(m):
        return (m & 7) * TOK + (m >> 3) * CH

    def gstart(m, s):
        pltpu.async_copy(vt_hbm.at[idx_v.at[pl.ds(chunk_base(m), CH)]],
                         bufs[s], gsems[s])

    gstart(0, 0)
    gstart(1, 1)

    def outer(ko, _):
        for par in range(NBUF):
            m = ko * NBUF + par

            @pl.when(lax.rem(m, DEPTH) == 0)
            def _():
                pc = m >> 3
                d = pc >> 1
                half = (pc & 1) * (CH // M)

                @plsc.parallel_loop(0, CH, unroll=2)
                def _(r):
                    rr = half + (r // M)
                    cc = lax.rem(r, M)
                    for j in range(NV):
                        sl = pl.ds(j * L, L)
                        pos_v[r, sl] = (row_v[rr, sl] + col_v[cc, sl]
                                        + deg_v[d, sl])

            s2_ = (par + 2) % NBUF

            @pl.when(m >= 2)
            def _():
                pltpu.make_async_copy(
                    bufs[s2_], out_hbm.at[pl.ds(0, CH)], ssems[s2_]).wait()

            @pl.when(m + 2 < nch)
            def _():
                gstart(m + 2, s2_)

            pltpu.make_async_copy(
                vt_hbm.at[idx_v.at[pl.ds(0, CH)]], bufs[par],
                gsems[par]).wait()

            @plsc.parallel_loop(0, CH, unroll=2)
            def _(r):
                for j in range(NV):
                    sl = pl.ds(j * L, L)
                    plsc.addupdate(bufs[par].at[r, sl], pos_v[r, sl])

            pltpu.async_copy(bufs[par],
                             out_hbm.at[pl.ds(wbase + chunk_base(m), CH)],
                             ssems[par])
        return 0

    lax.fori_loop(0, nch // NBUF, outer, 0)

    for s in (nch - 2) % NBUF, (nch - 1) % NBUF:
        pltpu.make_async_copy(
            bufs[s], out_hbm.at[pl.ds(0, CH)], ssems[s]).wait()


def kernel(x, value_emb, row_emb, col_emb, degree_emb):
    batch = x.shape[0]
    nrows = batch * TOK
    xf = x.reshape(nrows)
    vt = jnp.pad(value_emb, ((0, 1), (0, 0)))
    mesh = plsc.VectorSubcoreMesh(core_axis_name="c", subcore_axis_name="s")
    f = functools.partial(
        pl.kernel, mesh=mesh,
        out_type=jax.ShapeDtypeStruct((nrows, D_MODEL), jnp.float32),
        scratch_types=(
            [pltpu.VMEM((nrows // NW,), jnp.int32)]
            + [pltpu.VMEM((CH, D_MODEL), jnp.float32)] * NBUF
            + [pltpu.VMEM((M, D_MODEL), jnp.float32),
               pltpu.VMEM((M, D_MODEL), jnp.float32),
               pltpu.VMEM((MAX_DEGREE, D_MODEL), jnp.float32),
               pltpu.VMEM((CH, D_MODEL), jnp.float32)]
            + [pltpu.SemaphoreType.DMA] * (2 * NBUF)
        ),
    )(_sc_body)
    out = f(xf, vt, row_emb, col_emb, degree_emb)
    return out.reshape(batch, DEPTH, M * M, D_MODEL)


# speedup vs baseline: 4.6590x; 3.6434x over previous
import functools
import jax
import jax.numpy as jnp
from jax import lax
from jax.experimental import pallas as pl
from jax.experimental.pallas import tpu as pltpu
from jax.experimental.pallas import tpu_sc as plsc

P = 127
MAX_DEGREE = 8
M = 16
D_MODEL = 128
DEPTH = 8
TOK = DEPTH * M * M
NC, NS, L = 2, 16, 16
NW = NC * NS
CH = 128
NV = D_MODEL // L
NBUF = 4
BPW = 8


def _sc_body(x_hbm, vt_hbm, row_hbm, col_hbm, deg_hbm, out_hbm,
             idx_v, b0, b1, b2, b3, row_v, col_v, deg_v, pos_v, vt_sh,
             g0, g1, g2, g3, s0, s1, s2, s3):
    wid = lax.axis_index("s") * NC + lax.axis_index("c")

    @pl.when(lax.axis_index("s") == 0)
    def _():
        pltpu.sync_copy(vt_hbm, vt_sh)
    plsc.subcore_barrier()
    per_w = BPW * TOK
    nch = per_w // CH
    wbase = wid * per_w
    bufs = [b0, b1, b2, b3]
    gsems = [g0, g1, g2, g3]
    ssems = [s0, s1, s2, s3]

    pltpu.sync_copy(row_hbm, row_v)
    pltpu.sync_copy(col_hbm, col_v)
    pltpu.sync_copy(deg_hbm, deg_v)
    pltpu.sync_copy(x_hbm.at[pl.ds(wbase, per_w)], idx_v)

    def chunk_base(m):
        return (m & 7) * TOK + (m >> 3) * CH

    def gstart(m, s):
        pltpu.async_copy(vt_sh.at[idx_v.at[pl.ds(chunk_base(m), CH)]],
                         bufs[s], gsems[s])

    gstart(0, 0)
    gstart(1, 1)

    def outer(ko, _):
        for par in range(NBUF):
            m = ko * NBUF + par

            @pl.when(lax.rem(m, DEPTH) == 0)
            def _():
                pc = m >> 3
                d = pc >> 1
                half = (pc & 1) * (CH // M)

                @plsc.parallel_loop(0, CH, unroll=2)
                def _(r):
                    rr = half + (r // M)
                    cc = lax.rem(r, M)
                    for j in range(NV):
                        sl = pl.ds(j * L, L)
                        pos_v[r, sl] = (row_v[rr, sl] + col_v[cc, sl]
                                        + deg_v[d, sl])

            s2_ = (par + 2) % NBUF

            @pl.when(m >= 2)
            def _():
                pltpu.make_async_copy(
                    bufs[s2_], out_hbm.at[pl.ds(0, CH)], ssems[s2_]).wait()

            @pl.when(m + 2 < nch)
            def _():
                gstart(m + 2, s2_)

            pltpu.make_async_copy(
                vt_sh.at[idx_v.at[pl.ds(0, CH)]], bufs[par],
                gsems[par]).wait()

            @plsc.parallel_loop(0, CH, unroll=2)
            def _(r):
                for j in range(NV):
                    sl = pl.ds(j * L, L)
                    plsc.addupdate(bufs[par].at[r, sl], pos_v[r, sl])

            pltpu.async_copy(bufs[par],
                             out_hbm.at[pl.ds(wbase + chunk_base(m), CH)],
                             ssems[par])
        return 0

    lax.fori_loop(0, nch // NBUF, outer, 0)

    for s in (nch - 2) % NBUF, (nch - 1) % NBUF:
        pltpu.make_async_copy(
            bufs[s], out_hbm.at[pl.ds(0, CH)], ssems[s]).wait()


def kernel(x, value_emb, row_emb, col_emb, degree_emb):
    batch = x.shape[0]
    nrows = batch * TOK
    xf = x.reshape(nrows)
    vt = jnp.pad(value_emb, ((0, 1), (0, 0)))
    mesh = plsc.VectorSubcoreMesh(core_axis_name="c", subcore_axis_name="s")
    f = functools.partial(
        pl.kernel, mesh=mesh,
        out_type=jax.ShapeDtypeStruct((nrows, D_MODEL), jnp.float32),
        scratch_types=(
            [pltpu.VMEM((nrows // NW,), jnp.int32)]
            + [pltpu.VMEM((CH, D_MODEL), jnp.float32)] * NBUF
            + [pltpu.VMEM((M, D_MODEL), jnp.float32),
               pltpu.VMEM((M, D_MODEL), jnp.float32),
               pltpu.VMEM((MAX_DEGREE, D_MODEL), jnp.float32),
               pltpu.VMEM((CH, D_MODEL), jnp.float32),
               pltpu.VMEM_SHARED((P + 1, D_MODEL), jnp.float32)]
            + [pltpu.SemaphoreType.DMA] * (2 * NBUF)
        ),
    )(_sc_body)
    out = f(xf, vt, row_emb, col_emb, degree_emb)
    return out.reshape(batch, DEPTH, M * M, D_MODEL)
